# Initial kernel scaffold; baseline (speedup 1.0000x reference)
import jax
import jax.numpy as jnp
from jax.experimental import pallas as pl


def kernel(coords, box, pairs, q, p, t):
    raise NotImplementedError("write your pallas kernel here")


# speedup vs baseline: 5.2131x; 5.2131x over previous
import functools
import math

import jax
import jax.numpy as jnp
from jax import lax
from jax.experimental import pallas as pl
from jax.experimental.pallas import tpu as pltpu
from jax.experimental.pallas import tpu_sc as plsc

RANK2_CUTOFF = 10.0
ALPHA = 0.3
PREF = 1.0

N_ATOMS_P = 50000
N_PAIRS_P = 800000

_NC = 2
_NS = 16
_NW = _NC * _NS

_PER_W = N_PAIRS_P // _NW
_CH = 5000
_NCHUNK = _PER_W // _CH


def _gather_body(a_hbm, src_hbm, dst_hbm, gi_hbm, gj_hbm, idx_v, rows_v, sem):
    wid = lax.axis_index("s") * _NC + lax.axis_index("c")
    for side_idx, out_hbm in ((src_hbm, gi_hbm), (dst_hbm, gj_hbm)):
        for c in range(_NCHUNK):
            base = wid * _PER_W + c * _CH
            pltpu.sync_copy(side_idx.at[pl.ds(base, _CH)], idx_v)
            pltpu.async_copy(a_hbm.at[idx_v], rows_v, sem).wait()
            pltpu.sync_copy(rows_v, out_hbm.at[pl.ds(base, _CH)])


@functools.cache
def _sc_gather():
    return pl.kernel(
        _gather_body,
        out_type=[
            jax.ShapeDtypeStruct((N_PAIRS_P, 16), jnp.float32),
            jax.ShapeDtypeStruct((N_PAIRS_P, 16), jnp.float32),
        ],
        mesh=plsc.VectorSubcoreMesh(core_axis_name="c", subcore_axis_name="s"),
        scratch_types=[
            pltpu.VMEM((_CH,), jnp.int32),
            pltpu.VMEM((_CH, 16), jnp.float32),
            pltpu.SemaphoreType.DMA,
        ],
        compiler_params=pltpu.CompilerParams(use_tc_tiling_on_sc=False),
    )


def _energy_body(box_ref, binv_ref, gi_ref, gj_ref, out_ref):
    def row(ref, r):
        return ref[r, :].astype(jnp.float32)

    cix, ciy, ciz = row(gi_ref, 0), row(gi_ref, 1), row(gi_ref, 2)
    cjx, cjy, cjz = row(gj_ref, 0), row(gj_ref, 1), row(gj_ref, 2)

    def bf(v):
        return v.astype(jnp.bfloat16).astype(jnp.float32)

    d0, d1, d2 = bf(cjx - cix), bf(cjy - ciy), bf(cjz - ciz)

    def mat(ref, m, k):
        return bf(ref[m, k])

    s0 = d0 * mat(binv_ref, 0, 0) + d1 * mat(binv_ref, 1, 0) + d2 * mat(binv_ref, 2, 0)
    s1 = d0 * mat(binv_ref, 0, 1) + d1 * mat(binv_ref, 1, 1) + d2 * mat(binv_ref, 2, 1)
    s2 = d0 * mat(binv_ref, 0, 2) + d1 * mat(binv_ref, 1, 2) + d2 * mat(binv_ref, 2, 2)
    s0 = bf(s0 - jnp.round(s0))
    s1 = bf(s1 - jnp.round(s1))
    s2 = bf(s2 - jnp.round(s2))
    x = s0 * mat(box_ref, 0, 0) + s1 * mat(box_ref, 1, 0) + s2 * mat(box_ref, 2, 0)
    y = s0 * mat(box_ref, 0, 1) + s1 * mat(box_ref, 1, 1) + s2 * mat(box_ref, 2, 1)
    z = s0 * mat(box_ref, 0, 2) + s1 * mat(box_ref, 1, 2) + s2 * mat(box_ref, 2, 2)

    dr2 = x * x + y * y + z * z
    dr = jnp.sqrt(dr2)
    drInv = 1.0 / dr
    mask = dr <= RANK2_CUTOFF

    u = ALPHA * dr
    t = 1.0 / (1.0 + 0.3275911 * u)
    exp2u = jnp.exp(-u * u)
    erfc_u = (
        t
        * (0.254829592
           + t * (-0.284496736
                  + t * (1.421413741
                         + t * (-1.453152027 + t * 1.061405429))))
        * exp2u
    )
    u2 = u * u
    u3 = u2 * u
    u5 = u3 * u2
    u7 = u5 * u2
    prefpi = 2.0 / math.sqrt(math.pi)
    g = prefpi * exp2u
    f1 = erfc_u
    f3 = erfc_u + g * u
    f5 = erfc_u + g * ((3.0 * u + 2.0 * u3) / 3.0)
    f7 = erfc_u + g * ((15.0 * u + 10.0 * u3 + 4.0 * u5) / 15.0)
    f9 = erfc_u + g * ((8.0 * u7 + 28.0 * u5 + 70.0 * u3 + 105.0 * u) / 105.0)

    drInv2 = drInv * drInv
    D1 = drInv * f1
    drInv3 = drInv2 * drInv
    drInv5 = drInv3 * drInv2
    drInv7 = drInv5 * drInv2
    drInv9 = drInv7 * drInv2
    D3 = drInv3 * f3
    D5 = drInv5 * f5
    D7 = drInv7 * f7
    D9 = drInv9 * f9

    x2, y2, z2 = x * x, y * y, z * z
    xy, xz, yz = x * y, x * z, y * z

    tx, ty, tz = -x * D3, -y * D3, -z * D3
    txx = 3.0 * x2 * D5 - D3
    txy = 3.0 * xy * D5
    txz = 3.0 * xz * D5
    tyy = 3.0 * y2 * D5 - D3
    tyz = 3.0 * yz * D5
    tzz = 3.0 * z2 * D5 - D3
    txxx = -15.0 * x2 * x * D7 + 9.0 * x * D5
    txxy = -15.0 * x2 * y * D7 + 3.0 * y * D5
    txxz = -15.0 * x2 * z * D7 + 3.0 * z * D5
    tyyy = -15.0 * y2 * y * D7 + 9.0 * y * D5
    tyyx = -15.0 * y2 * x * D7 + 3.0 * x * D5
    tyyz = -15.0 * y2 * z * D7 + 3.0 * z * D5
    tzzz = -15.0 * z2 * z * D7 + 9.0 * z * D5
    tzzx = -15.0 * z2 * x * D7 + 3.0 * x * D5
    tzzy = -15.0 * z2 * y * D7 + 3.0 * y * D5
    txyz = -15.0 * x * y * z * D7
    txxxx = 105.0 * x2 * x2 * D9 - 90.0 * x2 * D7 + 9.0 * D5
    txxxy = 105.0 * x2 * xy * D9 - 45.0 * xy * D7
    txxxz = 105.0 * x2 * xz * D9 - 45.0 * xz * D7
    txxyy = 105.0 * x2 * y2 * D9 - 15.0 * (x2 + y2) * D7 + 3.0 * D5
    txxzz = 105.0 * x2 * z2 * D9 - 15.0 * (x2 + z2) * D7 + 3.0 * D5
    txxyz = 105.0 * x2 * yz * D9 - 15.0 * yz * D7
    tyyyy = 105.0 * y2 * y2 * D9 - 90.0 * y2 * D7 + 9.0 * D5
    tyyyx = 105.0 * y2 * xy * D9 - 45.0 * xy * D7
    tyyyz = 105.0 * y2 * yz * D9 - 45.0 * yz * D7
    tyyzz = 105.0 * y2 * z2 * D9 - 15.0 * (y2 + z2) * D7 + 3.0 * D5
    tyyxz = 105.0 * y2 * xz * D9 - 15.0 * xz * D7
    tzzzz = 105.0 * z2 * z2 * D9 - 90.0 * z2 * D7 + 9.0 * D5
    tzzzx = 105.0 * z2 * xz * D9 - 45.0 * xz * D7
    tzzzy = 105.0 * z2 * yz * D9 - 45.0 * yz * D7
    tzzxy = 105.0 * z2 * xy * D9 - 15.0 * xy * D7

    qi = row(gi_ref, 3)
    pix, piy, piz = row(gi_ref, 4), row(gi_ref, 5), row(gi_ref, 6)
    Qi0, Qi1, Qi2 = row(gi_ref, 7), row(gi_ref, 8), row(gi_ref, 9)
    Qi3, Qi4, Qi5 = row(gi_ref, 10), row(gi_ref, 11), row(gi_ref, 12)
    qj = row(gj_ref, 3)
    pjx, pjy, pjz = row(gj_ref, 4), row(gj_ref, 5), row(gj_ref, 6)
    Qj0, Qj1, Qj2 = row(gj_ref, 7), row(gj_ref, 8), row(gj_ref, 9)
    Qj3, Qj4, Qj5 = row(gj_ref, 10), row(gj_ref, 11), row(gj_ref, 12)

    r0 = (D1 * qj - tx * pjx - ty * pjy - tz * pjz
          + txx * Qj0 + txy * Qj1 + txz * Qj2 + tyy * Qj3 + tyz * Qj4 + tzz * Qj5)
    r1 = (tx * qj - txx * pjx - txy * pjy - txz * pjz
          + txxx * Qj0 + txxy * Qj1 + txxz * Qj2 + tyyx * Qj3 + txyz * Qj4 + tzzx * Qj5)
    r2 = (ty * qj - txy * pjx - tyy * pjy - tyz * pjz
          + txxy * Qj0 + tyyx * Qj1 + txyz * Qj2 + tyyy * Qj3 + tyyz * Qj4 + tzzy * Qj5)
    r3 = (tz * qj - txz * pjx - tyz * pjy - tzz * pjz
          + txxz * Qj0 + txyz * Qj1 + tzzx * Qj2 + tyyz * Qj3 + tzzy * Qj4 + tzzz * Qj5)
    r4 = (txx * qj - txxx * pjx - txxy * pjy - txxz * pjz
          + txxxx * Qj0 + txxxy * Qj1 + txxxz * Qj2 + txxyy * Qj3 + txxyz * Qj4 + txxzz * Qj5)
    r5 = (txy * qj - txxy * pjx - tyyx * pjy - txyz * pjz
          + txxxy * Qj0 + txxyy * Qj1 + txxyz * Qj2 + tyyyx * Qj3 + tyyxz * Qj4 + tzzxy * Qj5)
    r6 = (txz * qj - txxz * pjx - txyz * pjy - tzzx * pjz
          + txxxz * Qj0 + txxyz * Qj1 + txxzz * Qj2 + tyyxz * Qj3 + tzzxy * Qj4 + tzzzx * Qj5)
    r7 = (tyy * qj - tyyx * pjx - tyyy * pjy - tyyz * pjz
          + txxyy * Qj0 + tyyyx * Qj1 + tyyxz * Qj2 + tyyyy * Qj3 + tyyyz * Qj4 + tyyzz * Qj5)
    r8 = (tyz * qj - txyz * pjx - tyyz * pjy - tzzy * pjz
          + txxyz * Qj0 + tyyxz * Qj1 + tzzxy * Qj2 + tyyyz * Qj3 + tyyzz * Qj4 + tzzzy * Qj5)
    r9 = (tzz * qj - tzzx * pjx - tzzy * pjy - tzzz * pjz
          + txxzz * Qj0 + tzzxy * Qj1 + tzzzx * Qj2 + tyyzz * Qj3 + tzzzy * Qj4 + tzzzz * Qj5)

    ene = (qi * r0 + pix * r1 + piy * r2 + piz * r3
           + Qi0 * r4 + Qi1 * r5 + Qi2 * r6
           + Qi3 * r7 + Qi4 * r8 + Qi5 * r9)
    ene = jnp.where(mask, ene, 0.0)
    part = jnp.sum(ene)

    @pl.when(pl.program_id(0) == 0)
    def _():
        out_ref[0, 0] = 0.0

    out_ref[0, 0] += part


_TC_BLK = 6400
_TC_GRID = N_PAIRS_P // _TC_BLK


def _tc_energy(box, box_inv, giT, gjT):
    return pl.pallas_call(
        _energy_body,
        grid=(_TC_GRID,),
        in_specs=[
            pl.BlockSpec(memory_space=pltpu.SMEM),
            pl.BlockSpec(memory_space=pltpu.SMEM),
            pl.BlockSpec((16, _TC_BLK), lambda i: (0, i)),
            pl.BlockSpec((16, _TC_BLK), lambda i: (0, i)),
        ],
        out_specs=pl.BlockSpec(memory_space=pltpu.SMEM),
        out_shape=jax.ShapeDtypeStruct((1, 1), jnp.float32),
    )(box, box_inv, giT, gjT)


def kernel(coords, box, pairs, q, p, t):
    box_inv = jnp.linalg.inv(box)
    a_tab = jnp.concatenate(
        [
            coords,
            q[:, None],
            p,
            t[:, 0, 0][:, None] / 3,
            (t[:, 0, 1] + t[:, 1, 0])[:, None] / 3,
            (t[:, 0, 2] + t[:, 2, 0])[:, None] / 3,
            t[:, 1, 1][:, None] / 3,
            (t[:, 1, 2] + t[:, 2, 1])[:, None] / 3,
            t[:, 2, 2][:, None] / 3,
            jnp.zeros((coords.shape[0], 3), jnp.float32),
        ],
        axis=1,
    )
    src = pairs[:, 0]
    dst = pairs[:, 1]
    gi, gj = _sc_gather()(a_tab, src, dst)
    out = _tc_energy(box, box_inv, gi.T, gj.T)
    return PREF * out[0, 0]


# speedup vs baseline: 6.3608x; 1.2202x over previous
import functools
import math

import jax
import jax.numpy as jnp
from jax import lax
from jax.experimental import pallas as pl
from jax.experimental.pallas import tpu as pltpu
from jax.experimental.pallas import tpu_sc as plsc

RANK2_CUTOFF = 10.0
ALPHA = 0.3
PREF = 1.0

N_ATOMS_P = 50000
N_PAIRS_P = 800000

_NC = 2
_NS = 16
_NW = _NC * _NS

_PER_W = N_PAIRS_P // _NW
_CH = 5000
_NCHUNK = _PER_W // _CH


def _gather_body(a_hbm, src_hbm, dst_hbm, gi_hbm, gj_hbm, idx_v, rows_v, sem):
    wid = lax.axis_index("s") * _NC + lax.axis_index("c")
    for side_idx, out_hbm in ((src_hbm, gi_hbm), (dst_hbm, gj_hbm)):
        for c in range(_NCHUNK):
            base = wid * _PER_W + c * _CH
            pltpu.sync_copy(side_idx.at[pl.ds(base, _CH)], idx_v)
            pltpu.async_copy(a_hbm.at[idx_v], rows_v, sem).wait()
            pltpu.sync_copy(rows_v, out_hbm.at[pl.ds(base, _CH)])


@functools.cache
def _sc_gather():
    return pl.kernel(
        _gather_body,
        out_type=[
            jax.ShapeDtypeStruct((N_PAIRS_P, 16), jnp.float32),
            jax.ShapeDtypeStruct((N_PAIRS_P, 16), jnp.float32),
        ],
        mesh=plsc.VectorSubcoreMesh(core_axis_name="c", subcore_axis_name="s"),
        scratch_types=[
            pltpu.VMEM((_CH,), jnp.int32),
            pltpu.VMEM((_CH, 16), jnp.float32),
            pltpu.SemaphoreType.DMA,
        ],
        compiler_params=pltpu.CompilerParams(use_tc_tiling_on_sc=False),
    )


def _energy_body(box_ref, binv_ref, gi_ref, gj_ref, out_ref):
    giT = jnp.transpose(gi_ref[...], (1, 0))
    gjT = jnp.transpose(gj_ref[...], (1, 0))

    def row(arr, r):
        return arr[r, :]

    cix, ciy, ciz = row(giT, 0), row(giT, 1), row(giT, 2)
    cjx, cjy, cjz = row(gjT, 0), row(gjT, 1), row(gjT, 2)

    def bf(v):
        return v.astype(jnp.bfloat16).astype(jnp.float32)

    d0, d1, d2 = bf(cjx - cix), bf(cjy - ciy), bf(cjz - ciz)

    def mat(ref, m, k):
        return bf(ref[m, k])

    s0 = d0 * mat(binv_ref, 0, 0) + d1 * mat(binv_ref, 1, 0) + d2 * mat(binv_ref, 2, 0)
    s1 = d0 * mat(binv_ref, 0, 1) + d1 * mat(binv_ref, 1, 1) + d2 * mat(binv_ref, 2, 1)
    s2 = d0 * mat(binv_ref, 0, 2) + d1 * mat(binv_ref, 1, 2) + d2 * mat(binv_ref, 2, 2)
    s0 = bf(s0 - jnp.round(s0))
    s1 = bf(s1 - jnp.round(s1))
    s2 = bf(s2 - jnp.round(s2))
    x = s0 * mat(box_ref, 0, 0) + s1 * mat(box_ref, 1, 0) + s2 * mat(box_ref, 2, 0)
    y = s0 * mat(box_ref, 0, 1) + s1 * mat(box_ref, 1, 1) + s2 * mat(box_ref, 2, 1)
    z = s0 * mat(box_ref, 0, 2) + s1 * mat(box_ref, 1, 2) + s2 * mat(box_ref, 2, 2)

    dr2 = x * x + y * y + z * z
    dr = jnp.sqrt(dr2)
    drInv = 1.0 / dr
    mask = dr <= RANK2_CUTOFF

    u = ALPHA * dr
    t = 1.0 / (1.0 + 0.3275911 * u)
    exp2u = jnp.exp(-u * u)
    erfc_u = (
        t
        * (0.254829592
           + t * (-0.284496736
                  + t * (1.421413741
                         + t * (-1.453152027 + t * 1.061405429))))
        * exp2u
    )
    u2 = u * u
    u3 = u2 * u
    u5 = u3 * u2
    u7 = u5 * u2
    prefpi = 2.0 / math.sqrt(math.pi)
    g = prefpi * exp2u
    f1 = erfc_u
    f3 = erfc_u + g * u
    f5 = erfc_u + g * ((3.0 * u + 2.0 * u3) / 3.0)
    f7 = erfc_u + g * ((15.0 * u + 10.0 * u3 + 4.0 * u5) / 15.0)
    f9 = erfc_u + g * ((8.0 * u7 + 28.0 * u5 + 70.0 * u3 + 105.0 * u) / 105.0)

    drInv2 = drInv * drInv
    D1 = drInv * f1
    drInv3 = drInv2 * drInv
    drInv5 = drInv3 * drInv2
    drInv7 = drInv5 * drInv2
    drInv9 = drInv7 * drInv2
    D3 = drInv3 * f3
    D5 = drInv5 * f5
    D7 = drInv7 * f7
    D9 = drInv9 * f9

    x2, y2, z2 = x * x, y * y, z * z
    xy, xz, yz = x * y, x * z, y * z

    tx, ty, tz = -x * D3, -y * D3, -z * D3
    txx = 3.0 * x2 * D5 - D3
    txy = 3.0 * xy * D5
    txz = 3.0 * xz * D5
    tyy = 3.0 * y2 * D5 - D3
    tyz = 3.0 * yz * D5
    tzz = 3.0 * z2 * D5 - D3
    txxx = -15.0 * x2 * x * D7 + 9.0 * x * D5
    txxy = -15.0 * x2 * y * D7 + 3.0 * y * D5
    txxz = -15.0 * x2 * z * D7 + 3.0 * z * D5
    tyyy = -15.0 * y2 * y * D7 + 9.0 * y * D5
    tyyx = -15.0 * y2 * x * D7 + 3.0 * x * D5
    tyyz = -15.0 * y2 * z * D7 + 3.0 * z * D5
    tzzz = -15.0 * z2 * z * D7 + 9.0 * z * D5
    tzzx = -15.0 * z2 * x * D7 + 3.0 * x * D5
    tzzy = -15.0 * z2 * y * D7 + 3.0 * y * D5
    txyz = -15.0 * x * y * z * D7
    txxxx = 105.0 * x2 * x2 * D9 - 90.0 * x2 * D7 + 9.0 * D5
    txxxy = 105.0 * x2 * xy * D9 - 45.0 * xy * D7
    txxxz = 105.0 * x2 * xz * D9 - 45.0 * xz * D7
    txxyy = 105.0 * x2 * y2 * D9 - 15.0 * (x2 + y2) * D7 + 3.0 * D5
    txxzz = 105.0 * x2 * z2 * D9 - 15.0 * (x2 + z2) * D7 + 3.0 * D5
    txxyz = 105.0 * x2 * yz * D9 - 15.0 * yz * D7
    tyyyy = 105.0 * y2 * y2 * D9 - 90.0 * y2 * D7 + 9.0 * D5
    tyyyx = 105.0 * y2 * xy * D9 - 45.0 * xy * D7
    tyyyz = 105.0 * y2 * yz * D9 - 45.0 * yz * D7
    tyyzz = 105.0 * y2 * z2 * D9 - 15.0 * (y2 + z2) * D7 + 3.0 * D5
    tyyxz = 105.0 * y2 * xz * D9 - 15.0 * xz * D7
    tzzzz = 105.0 * z2 * z2 * D9 - 90.0 * z2 * D7 + 9.0 * D5
    tzzzx = 105.0 * z2 * xz * D9 - 45.0 * xz * D7
    tzzzy = 105.0 * z2 * yz * D9 - 45.0 * yz * D7
    tzzxy = 105.0 * z2 * xy * D9 - 15.0 * xy * D7

    qi = row(giT, 3)
    pix, piy, piz = row(giT, 4), row(giT, 5), row(giT, 6)
    Qi0, Qi1, Qi2 = row(giT, 7), row(giT, 8), row(giT, 9)
    Qi3, Qi4, Qi5 = row(giT, 10), row(giT, 11), row(giT, 12)
    qj = row(gjT, 3)
    pjx, pjy, pjz = row(gjT, 4), row(gjT, 5), row(gjT, 6)
    Qj0, Qj1, Qj2 = row(gjT, 7), row(gjT, 8), row(gjT, 9)
    Qj3, Qj4, Qj5 = row(gjT, 10), row(gjT, 11), row(gjT, 12)

    r0 = (D1 * qj - tx * pjx - ty * pjy - tz * pjz
          + txx * Qj0 + txy * Qj1 + txz * Qj2 + tyy * Qj3 + tyz * Qj4 + tzz * Qj5)
    r1 = (tx * qj - txx * pjx - txy * pjy - txz * pjz
          + txxx * Qj0 + txxy * Qj1 + txxz * Qj2 + tyyx * Qj3 + txyz * Qj4 + tzzx * Qj5)
    r2 = (ty * qj - txy * pjx - tyy * pjy - tyz * pjz
          + txxy * Qj0 + tyyx * Qj1 + txyz * Qj2 + tyyy * Qj3 + tyyz * Qj4 + tzzy * Qj5)
    r3 = (tz * qj - txz * pjx - tyz * pjy - tzz * pjz
          + txxz * Qj0 + txyz * Qj1 + tzzx * Qj2 + tyyz * Qj3 + tzzy * Qj4 + tzzz * Qj5)
    r4 = (txx * qj - txxx * pjx - txxy * pjy - txxz * pjz
          + txxxx * Qj0 + txxxy * Qj1 + txxxz * Qj2 + txxyy * Qj3 + txxyz * Qj4 + txxzz * Qj5)
    r5 = (txy * qj - txxy * pjx - tyyx * pjy - txyz * pjz
          + txxxy * Qj0 + txxyy * Qj1 + txxyz * Qj2 + tyyyx * Qj3 + tyyxz * Qj4 + tzzxy * Qj5)
    r6 = (txz * qj - txxz * pjx - txyz * pjy - tzzx * pjz
          + txxxz * Qj0 + txxyz * Qj1 + txxzz * Qj2 + tyyxz * Qj3 + tzzxy * Qj4 + tzzzx * Qj5)
    r7 = (tyy * qj - tyyx * pjx - tyyy * pjy - tyyz * pjz
          + txxyy * Qj0 + tyyyx * Qj1 + tyyxz * Qj2 + tyyyy * Qj3 + tyyyz * Qj4 + tyyzz * Qj5)
    r8 = (tyz * qj - txyz * pjx - tyyz * pjy - tzzy * pjz
          + txxyz * Qj0 + tyyxz * Qj1 + tzzxy * Qj2 + tyyyz * Qj3 + tyyzz * Qj4 + tzzzy * Qj5)
    r9 = (tzz * qj - tzzx * pjx - tzzy * pjy - tzzz * pjz
          + txxzz * Qj0 + tzzxy * Qj1 + tzzzx * Qj2 + tyyzz * Qj3 + tzzzy * Qj4 + tzzzz * Qj5)

    ene = (qi * r0 + pix * r1 + piy * r2 + piz * r3
           + Qi0 * r4 + Qi1 * r5 + Qi2 * r6
           + Qi3 * r7 + Qi4 * r8 + Qi5 * r9)
    ene = jnp.where(mask, ene, 0.0)
    part = jnp.sum(ene)

    @pl.when(pl.program_id(0) == 0)
    def _():
        out_ref[0, 0] = 0.0

    out_ref[0, 0] += part


_TC_BLK = 6400
_TC_GRID = N_PAIRS_P // _TC_BLK


def _tc_energy(box, box_inv, gi, gj):
    return pl.pallas_call(
        _energy_body,
        grid=(_TC_GRID,),
        in_specs=[
            pl.BlockSpec(memory_space=pltpu.SMEM),
            pl.BlockSpec(memory_space=pltpu.SMEM),
            pl.BlockSpec((_TC_BLK, 16), lambda i: (i, 0)),
            pl.BlockSpec((_TC_BLK, 16), lambda i: (i, 0)),
        ],
        out_specs=pl.BlockSpec(memory_space=pltpu.SMEM),
        out_shape=jax.ShapeDtypeStruct((1, 1), jnp.float32),
    )(box, box_inv, gi, gj)


def kernel(coords, box, pairs, q, p, t):
    box_inv = jnp.linalg.inv(box)
    a_tab = jnp.concatenate(
        [
            coords,
            q[:, None],
            p,
            t[:, 0, 0][:, None] / 3,
            (t[:, 0, 1] + t[:, 1, 0])[:, None] / 3,
            (t[:, 0, 2] + t[:, 2, 0])[:, None] / 3,
            t[:, 1, 1][:, None] / 3,
            (t[:, 1, 2] + t[:, 2, 1])[:, None] / 3,
            t[:, 2, 2][:, None] / 3,
            jnp.zeros((coords.shape[0], 3), jnp.float32),
        ],
        axis=1,
    )
    src = pairs[:, 0]
    dst = pairs[:, 1]
    gi, gj = _sc_gather()(a_tab, src, dst)
    out = _tc_energy(box, box_inv, gi, gj)
    return PREF * out[0, 0]


# speedup vs baseline: 10.3056x; 1.6202x over previous
import functools
import math

import jax
import jax.numpy as jnp
from jax import lax
from jax.experimental import pallas as pl
from jax.experimental.pallas import tpu as pltpu
from jax.experimental.pallas import tpu_sc as plsc

RANK2_CUTOFF = 10.0
ALPHA = 0.3
PREF = 1.0

N_ATOMS_P = 50000
N_PAIRS_P = 800000

_NC = 2
_NS = 16
_NW = _NC * _NS

_CH = 3200
_NCHUNKS_SIDE = N_PAIRS_P // _CH
_GROUPS = _CH // 16


_CH_ROWS = _CH // 128


def _gather_body(a_hbm, src_hbm, dst_hbm, gi_hbm, gj_hbm,
                 idx_v, rows_v, rowsT_v, sem):
    wid = lax.axis_index("s") * _NC + lax.axis_index("c")
    nt = (jnp.int32(_NCHUNKS_SIDE // _NW)
          + (wid < (_NCHUNKS_SIDE % _NW)).astype(jnp.int32))
    lanes = lax.iota(jnp.int32, 16)
    for side_idx, out_hbm in ((src_hbm, gi_hbm), (dst_hbm, gj_hbm)):

        def chunk_body(t, _):
            base = (wid + t * _NW) * _CH
            pltpu.sync_copy(side_idx.at[pl.ds(base, _CH)], idx_v)
            pltpu.async_copy(a_hbm.at[idx_v], rows_v, sem).wait()

            def grp_body(g, _):
                rows16 = g * 16 + lanes
                r = g // 8
                o = (g % 8) * 16
                for f in range(16):
                    v = plsc.load_gather(rows_v, [rows16, jnp.full((16,), f, jnp.int32)])
                    rowsT_v[f, r, pl.ds(o, 16)] = v
                return _

            lax.fori_loop(0, _GROUPS, grp_body, 0, unroll=False)
            c = wid + t * _NW
            pltpu.sync_copy(rowsT_v,
                            out_hbm.at[:, c // 2, pl.ds((c % 2) * _CH_ROWS, _CH_ROWS), :])
            return _

        lax.fori_loop(0, nt, chunk_body, 0, unroll=False)


@functools.cache
def _sc_gather():
    return pl.kernel(
        _gather_body,
        out_type=[
            jax.ShapeDtypeStruct((16, N_PAIRS_P // 6400, 50, 128), jnp.float32),
            jax.ShapeDtypeStruct((16, N_PAIRS_P // 6400, 50, 128), jnp.float32),
        ],
        mesh=plsc.VectorSubcoreMesh(core_axis_name="c", subcore_axis_name="s"),
        scratch_types=[
            pltpu.VMEM((_CH,), jnp.int32),
            pltpu.VMEM((_CH, 16), jnp.float32),
            pltpu.VMEM((16, _CH_ROWS, 128), jnp.float32),
            pltpu.SemaphoreType.DMA,
        ],
        compiler_params=pltpu.CompilerParams(
            use_tc_tiling_on_sc=False, needs_layout_passes=False),
    )


def _energy_body(box_ref, binv_ref, gi_ref, gj_ref, out_ref):
    giT = gi_ref
    gjT = gj_ref

    def row(arr, r):
        return arr[r, 0, :, :]

    cix, ciy, ciz = row(giT, 0), row(giT, 1), row(giT, 2)
    cjx, cjy, cjz = row(gjT, 0), row(gjT, 1), row(gjT, 2)

    def bf(v):
        return v.astype(jnp.bfloat16).astype(jnp.float32)

    d0, d1, d2 = bf(cjx - cix), bf(cjy - ciy), bf(cjz - ciz)

    def mat(ref, m, k):
        return bf(ref[m, k])

    s0 = d0 * mat(binv_ref, 0, 0) + d1 * mat(binv_ref, 1, 0) + d2 * mat(binv_ref, 2, 0)
    s1 = d0 * mat(binv_ref, 0, 1) + d1 * mat(binv_ref, 1, 1) + d2 * mat(binv_ref, 2, 1)
    s2 = d0 * mat(binv_ref, 0, 2) + d1 * mat(binv_ref, 1, 2) + d2 * mat(binv_ref, 2, 2)
    s0 = bf(s0 - jnp.round(s0))
    s1 = bf(s1 - jnp.round(s1))
    s2 = bf(s2 - jnp.round(s2))
    x = s0 * mat(box_ref, 0, 0) + s1 * mat(box_ref, 1, 0) + s2 * mat(box_ref, 2, 0)
    y = s0 * mat(box_ref, 0, 1) + s1 * mat(box_ref, 1, 1) + s2 * mat(box_ref, 2, 1)
    z = s0 * mat(box_ref, 0, 2) + s1 * mat(box_ref, 1, 2) + s2 * mat(box_ref, 2, 2)

    dr2 = x * x + y * y + z * z
    dr = jnp.sqrt(dr2)
    drInv = 1.0 / dr
    mask = dr <= RANK2_CUTOFF

    u = ALPHA * dr
    t = 1.0 / (1.0 + 0.3275911 * u)
    exp2u = jnp.exp(-u * u)
    erfc_u = (
        t
        * (0.254829592
           + t * (-0.284496736
                  + t * (1.421413741
                         + t * (-1.453152027 + t * 1.061405429))))
        * exp2u
    )
    u2 = u * u
    u3 = u2 * u
    u5 = u3 * u2
    u7 = u5 * u2
    prefpi = 2.0 / math.sqrt(math.pi)
    g = prefpi * exp2u
    f1 = erfc_u
    f3 = erfc_u + g * u
    f5 = erfc_u + g * ((3.0 * u + 2.0 * u3) / 3.0)
    f7 = erfc_u + g * ((15.0 * u + 10.0 * u3 + 4.0 * u5) / 15.0)
    f9 = erfc_u + g * ((8.0 * u7 + 28.0 * u5 + 70.0 * u3 + 105.0 * u) / 105.0)

    drInv2 = drInv * drInv
    D1 = drInv * f1
    drInv3 = drInv2 * drInv
    drInv5 = drInv3 * drInv2
    drInv7 = drInv5 * drInv2
    drInv9 = drInv7 * drInv2
    D3 = drInv3 * f3
    D5 = drInv5 * f5
    D7 = drInv7 * f7
    D9 = drInv9 * f9

    x2, y2, z2 = x * x, y * y, z * z
    xy, xz, yz = x * y, x * z, y * z

    tx, ty, tz = -x * D3, -y * D3, -z * D3
    txx = 3.0 * x2 * D5 - D3
    txy = 3.0 * xy * D5
    txz = 3.0 * xz * D5
    tyy = 3.0 * y2 * D5 - D3
    tyz = 3.0 * yz * D5
    tzz = 3.0 * z2 * D5 - D3
    txxx = -15.0 * x2 * x * D7 + 9.0 * x * D5
    txxy = -15.0 * x2 * y * D7 + 3.0 * y * D5
    txxz = -15.0 * x2 * z * D7 + 3.0 * z * D5
    tyyy = -15.0 * y2 * y * D7 + 9.0 * y * D5
    tyyx = -15.0 * y2 * x * D7 + 3.0 * x * D5
    tyyz = -15.0 * y2 * z * D7 + 3.0 * z * D5
    tzzz = -15.0 * z2 * z * D7 + 9.0 * z * D5
    tzzx = -15.0 * z2 * x * D7 + 3.0 * x * D5
    tzzy = -15.0 * z2 * y * D7 + 3.0 * y * D5
    txyz = -15.0 * x * y * z * D7
    txxxx = 105.0 * x2 * x2 * D9 - 90.0 * x2 * D7 + 9.0 * D5
    txxxy = 105.0 * x2 * xy * D9 - 45.0 * xy * D7
    txxxz = 105.0 * x2 * xz * D9 - 45.0 * xz * D7
    txxyy = 105.0 * x2 * y2 * D9 - 15.0 * (x2 + y2) * D7 + 3.0 * D5
    txxzz = 105.0 * x2 * z2 * D9 - 15.0 * (x2 + z2) * D7 + 3.0 * D5
    txxyz = 105.0 * x2 * yz * D9 - 15.0 * yz * D7
    tyyyy = 105.0 * y2 * y2 * D9 - 90.0 * y2 * D7 + 9.0 * D5
    tyyyx = 105.0 * y2 * xy * D9 - 45.0 * xy * D7
    tyyyz = 105.0 * y2 * yz * D9 - 45.0 * yz * D7
    tyyzz = 105.0 * y2 * z2 * D9 - 15.0 * (y2 + z2) * D7 + 3.0 * D5
    tyyxz = 105.0 * y2 * xz * D9 - 15.0 * xz * D7
    tzzzz = 105.0 * z2 * z2 * D9 - 90.0 * z2 * D7 + 9.0 * D5
    tzzzx = 105.0 * z2 * xz * D9 - 45.0 * xz * D7
    tzzzy = 105.0 * z2 * yz * D9 - 45.0 * yz * D7
    tzzxy = 105.0 * z2 * xy * D9 - 15.0 * xy * D7

    qi = row(giT, 3)
    pix, piy, piz = row(giT, 4), row(giT, 5), row(giT, 6)
    Qi0, Qi1, Qi2 = row(giT, 7), row(giT, 8), row(giT, 9)
    Qi3, Qi4, Qi5 = row(giT, 10), row(giT, 11), row(giT, 12)
    qj = row(gjT, 3)
    pjx, pjy, pjz = row(gjT, 4), row(gjT, 5), row(gjT, 6)
    Qj0, Qj1, Qj2 = row(gjT, 7), row(gjT, 8), row(gjT, 9)
    Qj3, Qj4, Qj5 = row(gjT, 10), row(gjT, 11), row(gjT, 12)

    r0 = (D1 * qj - tx * pjx - ty * pjy - tz * pjz
          + txx * Qj0 + txy * Qj1 + txz * Qj2 + tyy * Qj3 + tyz * Qj4 + tzz * Qj5)
    r1 = (tx * qj - txx * pjx - txy * pjy - txz * pjz
          + txxx * Qj0 + txxy * Qj1 + txxz * Qj2 + tyyx * Qj3 + txyz * Qj4 + tzzx * Qj5)
    r2 = (ty * qj - txy * pjx - tyy * pjy - tyz * pjz
          + txxy * Qj0 + tyyx * Qj1 + txyz * Qj2 + tyyy * Qj3 + tyyz * Qj4 + tzzy * Qj5)
    r3 = (tz * qj - txz * pjx - tyz * pjy - tzz * pjz
          + txxz * Qj0 + txyz * Qj1 + tzzx * Qj2 + tyyz * Qj3 + tzzy * Qj4 + tzzz * Qj5)
    r4 = (txx * qj - txxx * pjx - txxy * pjy - txxz * pjz
          + txxxx * Qj0 + txxxy * Qj1 + txxxz * Qj2 + txxyy * Qj3 + txxyz * Qj4 + txxzz * Qj5)
    r5 = (txy * qj - txxy * pjx - tyyx * pjy - txyz * pjz
          + txxxy * Qj0 + txxyy * Qj1 + txxyz * Qj2 + tyyyx * Qj3 + tyyxz * Qj4 + tzzxy * Qj5)
    r6 = (txz * qj - txxz * pjx - txyz * pjy - tzzx * pjz
          + txxxz * Qj0 + txxyz * Qj1 + txxzz * Qj2 + tyyxz * Qj3 + tzzxy * Qj4 + tzzzx * Qj5)
    r7 = (tyy * qj - tyyx * pjx - tyyy * pjy - tyyz * pjz
          + txxyy * Qj0 + tyyyx * Qj1 + tyyxz * Qj2 + tyyyy * Qj3 + tyyyz * Qj4 + tyyzz * Qj5)
    r8 = (tyz * qj - txyz * pjx - tyyz * pjy - tzzy * pjz
          + txxyz * Qj0 + tyyxz * Qj1 + tzzxy * Qj2 + tyyyz * Qj3 + tyyzz * Qj4 + tzzzy * Qj5)
    r9 = (tzz * qj - tzzx * pjx - tzzy * pjy - tzzz * pjz
          + txxzz * Qj0 + tzzxy * Qj1 + tzzzx * Qj2 + tyyzz * Qj3 + tzzzy * Qj4 + tzzzz * Qj5)

    ene = (qi * r0 + pix * r1 + piy * r2 + piz * r3
           + Qi0 * r4 + Qi1 * r5 + Qi2 * r6
           + Qi3 * r7 + Qi4 * r8 + Qi5 * r9)
    ene = jnp.where(mask, ene, 0.0)
    part = jnp.sum(ene)

    @pl.when(pl.program_id(0) == 0)
    def _():
        out_ref[0, 0] = 0.0

    out_ref[0, 0] += part


_TC_BLK = 6400
_TC_GRID = N_PAIRS_P // _TC_BLK


def _tc_energy(box, box_inv, gi, gj):
    return pl.pallas_call(
        _energy_body,
        grid=(_TC_GRID,),
        in_specs=[
            pl.BlockSpec(memory_space=pltpu.SMEM),
            pl.BlockSpec(memory_space=pltpu.SMEM),
            pl.BlockSpec((16, 1, 50, 128), lambda i: (0, i, 0, 0)),
            pl.BlockSpec((16, 1, 50, 128), lambda i: (0, i, 0, 0)),
        ],
        out_specs=pl.BlockSpec(memory_space=pltpu.SMEM),
        out_shape=jax.ShapeDtypeStruct((1, 1), jnp.float32),
    )(box, box_inv, gi, gj)


def kernel(coords, box, pairs, q, p, t):
    box_inv = jnp.linalg.inv(box)
    a_tab = jnp.concatenate(
        [
            coords,
            q[:, None],
            p,
            t[:, 0, 0][:, None] / 3,
            (t[:, 0, 1] + t[:, 1, 0])[:, None] / 3,
            (t[:, 0, 2] + t[:, 2, 0])[:, None] / 3,
            t[:, 1, 1][:, None] / 3,
            (t[:, 1, 2] + t[:, 2, 1])[:, None] / 3,
            t[:, 2, 2][:, None] / 3,
            jnp.zeros((coords.shape[0], 3), jnp.float32),
        ],
        axis=1,
    )
    src = pairs[:, 0]
    dst = pairs[:, 1]
    gi, gj = _sc_gather()(a_tab, src, dst)
    out = _tc_energy(box, box_inv, gi, gj)
    return PREF * out[0, 0]


# speedup vs baseline: 15.7510x; 1.5284x over previous
import functools
import math

import jax
import jax.numpy as jnp
from jax import lax
from jax.experimental import pallas as pl
from jax.experimental.pallas import tpu as pltpu
from jax.experimental.pallas import tpu_sc as plsc

RANK2_CUTOFF = 10.0
ALPHA = 0.3
PREF = 1.0

N_ATOMS_P = 50000
N_PAIRS_P = 800000

_NC = 2
_NS = 16
_NW = _NC * _NS

_CH = 3200
_NCHUNKS_SIDE = N_PAIRS_P // _CH
_GROUPS = _CH // 16


_CH_ROWS = _CH // 128


def _gather_body(a_hbm, src_hbm, dst_hbm, gi_hbm, gj_hbm,
                 idx_v, rows_v, rowsT_v, sem):
    wid = lax.axis_index("s") * _NC + lax.axis_index("c")
    nt = (jnp.int32(_NCHUNKS_SIDE // _NW)
          + (wid < (_NCHUNKS_SIDE % _NW)).astype(jnp.int32))
    lanes = lax.iota(jnp.int32, 16)
    for side_idx, out_hbm in ((src_hbm, gi_hbm), (dst_hbm, gj_hbm)):

        def chunk_body(t, _):
            base = (wid + t * _NW) * _CH
            pltpu.sync_copy(side_idx.at[pl.ds(base, _CH)], idx_v)
            pltpu.async_copy(a_hbm.at[idx_v], rows_v, sem).wait()

            @plsc.parallel_loop(0, _GROUPS, unroll=2)
            def grp_body(g):
                rows16 = g * 16 + lanes
                r = g // 8
                o = (g % 8) * 16
                for f in range(13):
                    v = plsc.load_gather(rows_v, [rows16, jnp.full((16,), f, jnp.int32)])
                    rowsT_v[f, r, pl.ds(o, 16)] = v
            c = wid + t * _NW
            pltpu.sync_copy(rowsT_v,
                            out_hbm.at[:, c // 2, pl.ds((c % 2) * _CH_ROWS, _CH_ROWS), :])
            return _

        lax.fori_loop(0, nt, chunk_body, 0, unroll=False)


@functools.cache
def _sc_gather():
    return pl.kernel(
        _gather_body,
        out_type=[
            jax.ShapeDtypeStruct((13, N_PAIRS_P // 6400, 50, 128), jnp.float32),
            jax.ShapeDtypeStruct((13, N_PAIRS_P // 6400, 50, 128), jnp.float32),
        ],
        mesh=plsc.VectorSubcoreMesh(core_axis_name="c", subcore_axis_name="s"),
        scratch_types=[
            pltpu.VMEM((_CH,), jnp.int32),
            pltpu.VMEM((_CH, 16), jnp.float32),
            pltpu.VMEM((13, _CH_ROWS, 128), jnp.float32),
            pltpu.SemaphoreType.DMA,
        ],
        compiler_params=pltpu.CompilerParams(
            use_tc_tiling_on_sc=False, needs_layout_passes=False),
    )


def _energy_body(box_ref, binv_ref, gi_ref, gj_ref, out_ref):
    giT = gi_ref
    gjT = gj_ref

    def row(arr, r):
        return arr[r, 0, :, :]

    cix, ciy, ciz = row(giT, 0), row(giT, 1), row(giT, 2)
    cjx, cjy, cjz = row(gjT, 0), row(gjT, 1), row(gjT, 2)

    def bf(v):
        return v.astype(jnp.bfloat16).astype(jnp.float32)

    d0, d1, d2 = bf(cjx - cix), bf(cjy - ciy), bf(cjz - ciz)

    def mat(ref, m, k):
        return bf(ref[m, k])

    s0 = d0 * mat(binv_ref, 0, 0) + d1 * mat(binv_ref, 1, 0) + d2 * mat(binv_ref, 2, 0)
    s1 = d0 * mat(binv_ref, 0, 1) + d1 * mat(binv_ref, 1, 1) + d2 * mat(binv_ref, 2, 1)
    s2 = d0 * mat(binv_ref, 0, 2) + d1 * mat(binv_ref, 1, 2) + d2 * mat(binv_ref, 2, 2)
    s0 = bf(s0 - jnp.round(s0))
    s1 = bf(s1 - jnp.round(s1))
    s2 = bf(s2 - jnp.round(s2))
    x = s0 * mat(box_ref, 0, 0) + s1 * mat(box_ref, 1, 0) + s2 * mat(box_ref, 2, 0)
    y = s0 * mat(box_ref, 0, 1) + s1 * mat(box_ref, 1, 1) + s2 * mat(box_ref, 2, 1)
    z = s0 * mat(box_ref, 0, 2) + s1 * mat(box_ref, 1, 2) + s2 * mat(box_ref, 2, 2)

    dr2 = x * x + y * y + z * z
    dr = jnp.sqrt(dr2)
    drInv = 1.0 / dr
    mask = dr <= RANK2_CUTOFF

    u = ALPHA * dr
    t = 1.0 / (1.0 + 0.3275911 * u)
    exp2u = jnp.exp(-u * u)
    erfc_u = (
        t
        * (0.254829592
           + t * (-0.284496736
                  + t * (1.421413741
                         + t * (-1.453152027 + t * 1.061405429))))
        * exp2u
    )
    u2 = u * u
    u3 = u2 * u
    u5 = u3 * u2
    u7 = u5 * u2
    prefpi = 2.0 / math.sqrt(math.pi)
    g = prefpi * exp2u
    f1 = erfc_u
    f3 = erfc_u + g * u
    f5 = erfc_u + g * ((3.0 * u + 2.0 * u3) / 3.0)
    f7 = erfc_u + g * ((15.0 * u + 10.0 * u3 + 4.0 * u5) / 15.0)
    f9 = erfc_u + g * ((8.0 * u7 + 28.0 * u5 + 70.0 * u3 + 105.0 * u) / 105.0)

    drInv2 = drInv * drInv
    D1 = drInv * f1
    drInv3 = drInv2 * drInv
    drInv5 = drInv3 * drInv2
    drInv7 = drInv5 * drInv2
    drInv9 = drInv7 * drInv2
    D3 = drInv3 * f3
    D5 = drInv5 * f5
    D7 = drInv7 * f7
    D9 = drInv9 * f9

    x2, y2, z2 = x * x, y * y, z * z
    xy, xz, yz = x * y, x * z, y * z

    tx, ty, tz = -x * D3, -y * D3, -z * D3
    txx = 3.0 * x2 * D5 - D3
    txy = 3.0 * xy * D5
    txz = 3.0 * xz * D5
    tyy = 3.0 * y2 * D5 - D3
    tyz = 3.0 * yz * D5
    tzz = 3.0 * z2 * D5 - D3
    txxx = -15.0 * x2 * x * D7 + 9.0 * x * D5
    txxy = -15.0 * x2 * y * D7 + 3.0 * y * D5
    txxz = -15.0 * x2 * z * D7 + 3.0 * z * D5
    tyyy = -15.0 * y2 * y * D7 + 9.0 * y * D5
    tyyx = -15.0 * y2 * x * D7 + 3.0 * x * D5
    tyyz = -15.0 * y2 * z * D7 + 3.0 * z * D5
    tzzz = -15.0 * z2 * z * D7 + 9.0 * z * D5
    tzzx = -15.0 * z2 * x * D7 + 3.0 * x * D5
    tzzy = -15.0 * z2 * y * D7 + 3.0 * y * D5
    txyz = -15.0 * x * y * z * D7
    txxxx = 105.0 * x2 * x2 * D9 - 90.0 * x2 * D7 + 9.0 * D5
    txxxy = 105.0 * x2 * xy * D9 - 45.0 * xy * D7
    txxxz = 105.0 * x2 * xz * D9 - 45.0 * xz * D7
    txxyy = 105.0 * x2 * y2 * D9 - 15.0 * (x2 + y2) * D7 + 3.0 * D5
    txxzz = 105.0 * x2 * z2 * D9 - 15.0 * (x2 + z2) * D7 + 3.0 * D5
    txxyz = 105.0 * x2 * yz * D9 - 15.0 * yz * D7
    tyyyy = 105.0 * y2 * y2 * D9 - 90.0 * y2 * D7 + 9.0 * D5
    tyyyx = 105.0 * y2 * xy * D9 - 45.0 * xy * D7
    tyyyz = 105.0 * y2 * yz * D9 - 45.0 * yz * D7
    tyyzz = 105.0 * y2 * z2 * D9 - 15.0 * (y2 + z2) * D7 + 3.0 * D5
    tyyxz = 105.0 * y2 * xz * D9 - 15.0 * xz * D7
    tzzzz = 105.0 * z2 * z2 * D9 - 90.0 * z2 * D7 + 9.0 * D5
    tzzzx = 105.0 * z2 * xz * D9 - 45.0 * xz * D7
    tzzzy = 105.0 * z2 * yz * D9 - 45.0 * yz * D7
    tzzxy = 105.0 * z2 * xy * D9 - 15.0 * xy * D7

    qi = row(giT, 3)
    pix, piy, piz = row(giT, 4), row(giT, 5), row(giT, 6)
    Qi0, Qi1, Qi2 = row(giT, 7), row(giT, 8), row(giT, 9)
    Qi3, Qi4, Qi5 = row(giT, 10), row(giT, 11), row(giT, 12)
    qj = row(gjT, 3)
    pjx, pjy, pjz = row(gjT, 4), row(gjT, 5), row(gjT, 6)
    Qj0, Qj1, Qj2 = row(gjT, 7), row(gjT, 8), row(gjT, 9)
    Qj3, Qj4, Qj5 = row(gjT, 10), row(gjT, 11), row(gjT, 12)

    r0 = (D1 * qj - tx * pjx - ty * pjy - tz * pjz
          + txx * Qj0 + txy * Qj1 + txz * Qj2 + tyy * Qj3 + tyz * Qj4 + tzz * Qj5)
    r1 = (tx * qj - txx * pjx - txy * pjy - txz * pjz
          + txxx * Qj0 + txxy * Qj1 + txxz * Qj2 + tyyx * Qj3 + txyz * Qj4 + tzzx * Qj5)
    r2 = (ty * qj - txy * pjx - tyy * pjy - tyz * pjz
          + txxy * Qj0 + tyyx * Qj1 + txyz * Qj2 + tyyy * Qj3 + tyyz * Qj4 + tzzy * Qj5)
    r3 = (tz * qj - txz * pjx - tyz * pjy - tzz * pjz
          + txxz * Qj0 + txyz * Qj1 + tzzx * Qj2 + tyyz * Qj3 + tzzy * Qj4 + tzzz * Qj5)
    r4 = (txx * qj - txxx * pjx - txxy * pjy - txxz * pjz
          + txxxx * Qj0 + txxxy * Qj1 + txxxz * Qj2 + txxyy * Qj3 + txxyz * Qj4 + txxzz * Qj5)
    r5 = (txy * qj - txxy * pjx - tyyx * pjy - txyz * pjz
          + txxxy * Qj0 + txxyy * Qj1 + txxyz * Qj2 + tyyyx * Qj3 + tyyxz * Qj4 + tzzxy * Qj5)
    r6 = (txz * qj - txxz * pjx - txyz * pjy - tzzx * pjz
          + txxxz * Qj0 + txxyz * Qj1 + txxzz * Qj2 + tyyxz * Qj3 + tzzxy * Qj4 + tzzzx * Qj5)
    r7 = (tyy * qj - tyyx * pjx - tyyy * pjy - tyyz * pjz
          + txxyy * Qj0 + tyyyx * Qj1 + tyyxz * Qj2 + tyyyy * Qj3 + tyyyz * Qj4 + tyyzz * Qj5)
    r8 = (tyz * qj - txyz * pjx - tyyz * pjy - tzzy * pjz
          + txxyz * Qj0 + tyyxz * Qj1 + tzzxy * Qj2 + tyyyz * Qj3 + tyyzz * Qj4 + tzzzy * Qj5)
    r9 = (tzz * qj - tzzx * pjx - tzzy * pjy - tzzz * pjz
          + txxzz * Qj0 + tzzxy * Qj1 + tzzzx * Qj2 + tyyzz * Qj3 + tzzzy * Qj4 + tzzzz * Qj5)

    ene = (qi * r0 + pix * r1 + piy * r2 + piz * r3
           + Qi0 * r4 + Qi1 * r5 + Qi2 * r6
           + Qi3 * r7 + Qi4 * r8 + Qi5 * r9)
    ene = jnp.where(mask, ene, 0.0)
    part = jnp.sum(ene)

    @pl.when(pl.program_id(0) == 0)
    def _():
        out_ref[0, 0] = 0.0

    out_ref[0, 0] += part


_TC_BLK = 6400
_TC_GRID = N_PAIRS_P // _TC_BLK


def _tc_energy(box, box_inv, gi, gj):
    return pl.pallas_call(
        _energy_body,
        grid=(_TC_GRID,),
        in_specs=[
            pl.BlockSpec(memory_space=pltpu.SMEM),
            pl.BlockSpec(memory_space=pltpu.SMEM),
            pl.BlockSpec((13, 1, 50, 128), lambda i: (0, i, 0, 0)),
            pl.BlockSpec((13, 1, 50, 128), lambda i: (0, i, 0, 0)),
        ],
        out_specs=pl.BlockSpec(memory_space=pltpu.SMEM),
        out_shape=jax.ShapeDtypeStruct((1, 1), jnp.float32),
    )(box, box_inv, gi, gj)


def kernel(coords, box, pairs, q, p, t):
    box_inv = jnp.linalg.inv(box)
    a_tab = jnp.concatenate(
        [
            coords,
            q[:, None],
            p,
            t[:, 0, 0][:, None] / 3,
            (t[:, 0, 1] + t[:, 1, 0])[:, None] / 3,
            (t[:, 0, 2] + t[:, 2, 0])[:, None] / 3,
            t[:, 1, 1][:, None] / 3,
            (t[:, 1, 2] + t[:, 2, 1])[:, None] / 3,
            t[:, 2, 2][:, None] / 3,
            jnp.zeros((coords.shape[0], 3), jnp.float32),
        ],
        axis=1,
    )
    src = pairs[:, 0]
    dst = pairs[:, 1]
    gi, gj = _sc_gather()(a_tab, src, dst)
    out = _tc_energy(box, box_inv, gi, gj)
    return PREF * out[0, 0]


# speedup vs baseline: 16.9583x; 1.0766x over previous
import functools
import math

import jax
import jax.numpy as jnp
from jax import lax
from jax.experimental import pallas as pl
from jax.experimental.pallas import tpu as pltpu
from jax.experimental.pallas import tpu_sc as plsc

RANK2_CUTOFF = 10.0
ALPHA = 0.3
PREF = 1.0

N_ATOMS_P = 50000
N_PAIRS_P = 800000

_NC = 2
_NS = 16
_NW = _NC * _NS

_CH = 1280
_NCHUNKS_SIDE = N_PAIRS_P // _CH
_GROUPS = _CH // 16
_CH_ROWS = _CH // 128
_CPM = 6400 // _CH
_T_PER_W = (_NCHUNKS_SIDE + _NW - 1) // _NW


def _gather_body(a_hbm, src_hbm, dst_hbm, gi_hbm, gj_hbm,
                 idx0_v, idx1_v, rows0_v, rows1_v, rowsT_v, sem0, sem1):
    wid = lax.axis_index("s") * _NC + lax.axis_index("c")
    lanes = lax.iota(jnp.int32, 16)
    idx_b = (idx0_v, idx1_v)
    rows_b = (rows0_v, rows1_v)
    sem_b = (sem0, sem1)

    def issue(side_idx, t, par):
        c = wid + t * _NW

        @pl.when(c < _NCHUNKS_SIDE)
        def _():
            pltpu.sync_copy(side_idx.at[pl.ds(c * _CH, _CH)], idx_b[par])
            pltpu.async_copy(a_hbm.at[idx_b[par]], rows_b[par], sem_b[par])

    for side_idx, out_hbm in ((src_hbm, gi_hbm), (dst_hbm, gj_hbm)):
        issue(side_idx, 0, 0)
        for t in range(_T_PER_W):
            par = t % 2
            if t + 1 < _T_PER_W:
                issue(side_idx, t + 1, (t + 1) % 2)
            c = wid + t * _NW

            @pl.when(c < _NCHUNKS_SIDE)
            def _():
                pltpu.make_async_copy(
                    a_hbm.at[idx_b[par]], rows_b[par], sem_b[par]).wait()
                rows_v = rows_b[par]

                @plsc.parallel_loop(0, _GROUPS, unroll=2)
                def grp_body(g):
                    rows16 = g * 16 + lanes
                    r = g // 8
                    o = (g % 8) * 16
                    for f in range(13):
                        v = plsc.load_gather(
                            rows_v, [rows16, jnp.full((16,), f, jnp.int32)])
                        rowsT_v[f, r, pl.ds(o, 16)] = v

                pltpu.sync_copy(
                    rowsT_v,
                    out_hbm.at[:, c // _CPM,
                               pl.ds((c % _CPM) * _CH_ROWS, _CH_ROWS), :])


@functools.cache
def _sc_gather():
    return pl.kernel(
        _gather_body,
        out_type=[
            jax.ShapeDtypeStruct((13, N_PAIRS_P // 6400, 50, 128), jnp.float32),
            jax.ShapeDtypeStruct((13, N_PAIRS_P // 6400, 50, 128), jnp.float32),
        ],
        mesh=plsc.VectorSubcoreMesh(core_axis_name="c", subcore_axis_name="s"),
        scratch_types=[
            pltpu.VMEM((_CH,), jnp.int32),
            pltpu.VMEM((_CH,), jnp.int32),
            pltpu.VMEM((_CH, 16), jnp.float32),
            pltpu.VMEM((_CH, 16), jnp.float32),
            pltpu.VMEM((13, _CH_ROWS, 128), jnp.float32),
            pltpu.SemaphoreType.DMA,
            pltpu.SemaphoreType.DMA,
        ],
        compiler_params=pltpu.CompilerParams(
            use_tc_tiling_on_sc=False, needs_layout_passes=False),
    )


def _energy_body(box_ref, binv_ref, gi_ref, gj_ref, out_ref):
    giT = gi_ref
    gjT = gj_ref

    def row(arr, r):
        return arr[r, 0, :, :]

    cix, ciy, ciz = row(giT, 0), row(giT, 1), row(giT, 2)
    cjx, cjy, cjz = row(gjT, 0), row(gjT, 1), row(gjT, 2)

    def bf(v):
        return v.astype(jnp.bfloat16).astype(jnp.float32)

    d0, d1, d2 = bf(cjx - cix), bf(cjy - ciy), bf(cjz - ciz)

    def mat(ref, m, k):
        return bf(ref[m, k])

    s0 = d0 * mat(binv_ref, 0, 0) + d1 * mat(binv_ref, 1, 0) + d2 * mat(binv_ref, 2, 0)
    s1 = d0 * mat(binv_ref, 0, 1) + d1 * mat(binv_ref, 1, 1) + d2 * mat(binv_ref, 2, 1)
    s2 = d0 * mat(binv_ref, 0, 2) + d1 * mat(binv_ref, 1, 2) + d2 * mat(binv_ref, 2, 2)
    s0 = bf(s0 - jnp.round(s0))
    s1 = bf(s1 - jnp.round(s1))
    s2 = bf(s2 - jnp.round(s2))
    x = s0 * mat(box_ref, 0, 0) + s1 * mat(box_ref, 1, 0) + s2 * mat(box_ref, 2, 0)
    y = s0 * mat(box_ref, 0, 1) + s1 * mat(box_ref, 1, 1) + s2 * mat(box_ref, 2, 1)
    z = s0 * mat(box_ref, 0, 2) + s1 * mat(box_ref, 1, 2) + s2 * mat(box_ref, 2, 2)

    dr2 = x * x + y * y + z * z
    dr = jnp.sqrt(dr2)
    drInv = 1.0 / dr
    mask = dr <= RANK2_CUTOFF

    u = ALPHA * dr
    t = 1.0 / (1.0 + 0.3275911 * u)
    exp2u = jnp.exp(-u * u)
    erfc_u = (
        t
        * (0.254829592
           + t * (-0.284496736
                  + t * (1.421413741
                         + t * (-1.453152027 + t * 1.061405429))))
        * exp2u
    )
    u2 = u * u
    u3 = u2 * u
    u5 = u3 * u2
    u7 = u5 * u2
    prefpi = 2.0 / math.sqrt(math.pi)
    g = prefpi * exp2u
    f1 = erfc_u
    f3 = erfc_u + g * u
    f5 = erfc_u + g * ((3.0 * u + 2.0 * u3) / 3.0)
    f7 = erfc_u + g * ((15.0 * u + 10.0 * u3 + 4.0 * u5) / 15.0)
    f9 = erfc_u + g * ((8.0 * u7 + 28.0 * u5 + 70.0 * u3 + 105.0 * u) / 105.0)

    drInv2 = drInv * drInv
    D1 = drInv * f1
    drInv3 = drInv2 * drInv
    drInv5 = drInv3 * drInv2
    drInv7 = drInv5 * drInv2
    drInv9 = drInv7 * drInv2
    D3 = drInv3 * f3
    D5 = drInv5 * f5
    D7 = drInv7 * f7
    D9 = drInv9 * f9

    x2, y2, z2 = x * x, y * y, z * z
    xy, xz, yz = x * y, x * z, y * z

    tx, ty, tz = -x * D3, -y * D3, -z * D3
    txx = 3.0 * x2 * D5 - D3
    txy = 3.0 * xy * D5
    txz = 3.0 * xz * D5
    tyy = 3.0 * y2 * D5 - D3
    tyz = 3.0 * yz * D5
    tzz = 3.0 * z2 * D5 - D3
    txxx = -15.0 * x2 * x * D7 + 9.0 * x * D5
    txxy = -15.0 * x2 * y * D7 + 3.0 * y * D5
    txxz = -15.0 * x2 * z * D7 + 3.0 * z * D5
    tyyy = -15.0 * y2 * y * D7 + 9.0 * y * D5
    tyyx = -15.0 * y2 * x * D7 + 3.0 * x * D5
    tyyz = -15.0 * y2 * z * D7 + 3.0 * z * D5
    tzzz = -15.0 * z2 * z * D7 + 9.0 * z * D5
    tzzx = -15.0 * z2 * x * D7 + 3.0 * x * D5
    tzzy = -15.0 * z2 * y * D7 + 3.0 * y * D5
    txyz = -15.0 * x * y * z * D7
    txxxx = 105.0 * x2 * x2 * D9 - 90.0 * x2 * D7 + 9.0 * D5
    txxxy = 105.0 * x2 * xy * D9 - 45.0 * xy * D7
    txxxz = 105.0 * x2 * xz * D9 - 45.0 * xz * D7
    txxyy = 105.0 * x2 * y2 * D9 - 15.0 * (x2 + y2) * D7 + 3.0 * D5
    txxzz = 105.0 * x2 * z2 * D9 - 15.0 * (x2 + z2) * D7 + 3.0 * D5
    txxyz = 105.0 * x2 * yz * D9 - 15.0 * yz * D7
    tyyyy = 105.0 * y2 * y2 * D9 - 90.0 * y2 * D7 + 9.0 * D5
    tyyyx = 105.0 * y2 * xy * D9 - 45.0 * xy * D7
    tyyyz = 105.0 * y2 * yz * D9 - 45.0 * yz * D7
    tyyzz = 105.0 * y2 * z2 * D9 - 15.0 * (y2 + z2) * D7 + 3.0 * D5
    tyyxz = 105.0 * y2 * xz * D9 - 15.0 * xz * D7
    tzzzz = 105.0 * z2 * z2 * D9 - 90.0 * z2 * D7 + 9.0 * D5
    tzzzx = 105.0 * z2 * xz * D9 - 45.0 * xz * D7
    tzzzy = 105.0 * z2 * yz * D9 - 45.0 * yz * D7
    tzzxy = 105.0 * z2 * xy * D9 - 15.0 * xy * D7

    qi = row(giT, 3)
    pix, piy, piz = row(giT, 4), row(giT, 5), row(giT, 6)
    Qi0, Qi1, Qi2 = row(giT, 7), row(giT, 8), row(giT, 9)
    Qi3, Qi4, Qi5 = row(giT, 10), row(giT, 11), row(giT, 12)
    qj = row(gjT, 3)
    pjx, pjy, pjz = row(gjT, 4), row(gjT, 5), row(gjT, 6)
    Qj0, Qj1, Qj2 = row(gjT, 7), row(gjT, 8), row(gjT, 9)
    Qj3, Qj4, Qj5 = row(gjT, 10), row(gjT, 11), row(gjT, 12)

    r0 = (D1 * qj - tx * pjx - ty * pjy - tz * pjz
          + txx * Qj0 + txy * Qj1 + txz * Qj2 + tyy * Qj3 + tyz * Qj4 + tzz * Qj5)
    r1 = (tx * qj - txx * pjx - txy * pjy - txz * pjz
          + txxx * Qj0 + txxy * Qj1 + txxz * Qj2 + tyyx * Qj3 + txyz * Qj4 + tzzx * Qj5)
    r2 = (ty * qj - txy * pjx - tyy * pjy - tyz * pjz
          + txxy * Qj0 + tyyx * Qj1 + txyz * Qj2 + tyyy * Qj3 + tyyz * Qj4 + tzzy * Qj5)
    r3 = (tz * qj - txz * pjx - tyz * pjy - tzz * pjz
          + txxz * Qj0 + txyz * Qj1 + tzzx * Qj2 + tyyz * Qj3 + tzzy * Qj4 + tzzz * Qj5)
    r4 = (txx * qj - txxx * pjx - txxy * pjy - txxz * pjz
          + txxxx * Qj0 + txxxy * Qj1 + txxxz * Qj2 + txxyy * Qj3 + txxyz * Qj4 + txxzz * Qj5)
    r5 = (txy * qj - txxy * pjx - tyyx * pjy - txyz * pjz
          + txxxy * Qj0 + txxyy * Qj1 + txxyz * Qj2 + tyyyx * Qj3 + tyyxz * Qj4 + tzzxy * Qj5)
    r6 = (txz * qj - txxz * pjx - txyz * pjy - tzzx * pjz
          + txxxz * Qj0 + txxyz * Qj1 + txxzz * Qj2 + tyyxz * Qj3 + tzzxy * Qj4 + tzzzx * Qj5)
    r7 = (tyy * qj - tyyx * pjx - tyyy * pjy - tyyz * pjz
          + txxyy * Qj0 + tyyyx * Qj1 + tyyxz * Qj2 + tyyyy * Qj3 + tyyyz * Qj4 + tyyzz * Qj5)
    r8 = (tyz * qj - txyz * pjx - tyyz * pjy - tzzy * pjz
          + txxyz * Qj0 + tyyxz * Qj1 + tzzxy * Qj2 + tyyyz * Qj3 + tyyzz * Qj4 + tzzzy * Qj5)
    r9 = (tzz * qj - tzzx * pjx - tzzy * pjy - tzzz * pjz
          + txxzz * Qj0 + tzzxy * Qj1 + tzzzx * Qj2 + tyyzz * Qj3 + tzzzy * Qj4 + tzzzz * Qj5)

    ene = (qi * r0 + pix * r1 + piy * r2 + piz * r3
           + Qi0 * r4 + Qi1 * r5 + Qi2 * r6
           + Qi3 * r7 + Qi4 * r8 + Qi5 * r9)
    ene = jnp.where(mask, ene, 0.0)
    part = jnp.sum(ene)

    @pl.when(pl.program_id(0) == 0)
    def _():
        out_ref[0, 0] = 0.0

    out_ref[0, 0] += part


_TC_BLK = 6400
_TC_GRID = N_PAIRS_P // _TC_BLK


def _tc_energy(box, box_inv, gi, gj):
    return pl.pallas_call(
        _energy_body,
        grid=(_TC_GRID,),
        in_specs=[
            pl.BlockSpec(memory_space=pltpu.SMEM),
            pl.BlockSpec(memory_space=pltpu.SMEM),
            pl.BlockSpec((13, 1, 50, 128), lambda i: (0, i, 0, 0)),
            pl.BlockSpec((13, 1, 50, 128), lambda i: (0, i, 0, 0)),
        ],
        out_specs=pl.BlockSpec(memory_space=pltpu.SMEM),
        out_shape=jax.ShapeDtypeStruct((1, 1), jnp.float32),
    )(box, box_inv, gi, gj)


def kernel(coords, box, pairs, q, p, t):
    box_inv = jnp.linalg.inv(box)
    a_tab = jnp.concatenate(
        [
            coords,
            q[:, None],
            p,
            t[:, 0, 0][:, None] / 3,
            (t[:, 0, 1] + t[:, 1, 0])[:, None] / 3,
            (t[:, 0, 2] + t[:, 2, 0])[:, None] / 3,
            t[:, 1, 1][:, None] / 3,
            (t[:, 1, 2] + t[:, 2, 1])[:, None] / 3,
            t[:, 2, 2][:, None] / 3,
            jnp.zeros((coords.shape[0], 3), jnp.float32),
        ],
        axis=1,
    )
    src = pairs[:, 0]
    dst = pairs[:, 1]
    gi, gj = _sc_gather()(a_tab, src, dst)
    out = _tc_energy(box, box_inv, gi, gj)
    return PREF * out[0, 0]


# speedup vs baseline: 18.3064x; 1.0795x over previous
import functools
import math

import jax
import jax.numpy as jnp
from jax import lax
from jax.experimental import pallas as pl
from jax.experimental.pallas import tpu as pltpu
from jax.experimental.pallas import tpu_sc as plsc

RANK2_CUTOFF = 10.0
ALPHA = 0.3
PREF = 1.0

N_ATOMS_P = 50000
N_PAIRS_P = 800000

_NC = 2
_NS = 16
_NW = _NC * _NS

_CH = 1280
_NCHUNKS_SIDE = N_PAIRS_P // _CH
_GROUPS = _CH // 16
_CH_ROWS = _CH // 128
_CPM = 6400 // _CH
_T_PER_W = (_NCHUNKS_SIDE + _NW - 1) // _NW


def _make_gather_body(n_pairs):
  n_chunks = n_pairs // _CH
  t_per_w = (n_chunks + _NW - 1) // _NW

  def _gather_body(a_hbm, src_hbm, dst_hbm, gi_hbm, gj_hbm,
                   idx0_v, idx1_v, rows0_v, rows1_v, rowsT_v, sem0, sem1):
    wid = lax.axis_index("s") * _NC + lax.axis_index("c")
    lanes = lax.iota(jnp.int32, 16)
    idx_b = (idx0_v, idx1_v)
    rows_b = (rows0_v, rows1_v)
    sem_b = (sem0, sem1)

    def issue(side_idx, t, par):
        c = wid + t * _NW

        @pl.when(c < n_chunks)
        def _():
            pltpu.sync_copy(side_idx.at[pl.ds(c * _CH, _CH)], idx_b[par])
            pltpu.async_copy(a_hbm.at[idx_b[par]], rows_b[par], sem_b[par])

    for side_idx, out_hbm in ((src_hbm, gi_hbm), (dst_hbm, gj_hbm)):
        issue(side_idx, 0, 0)
        for t in range(t_per_w):
            par = t % 2
            if t + 1 < t_per_w:
                issue(side_idx, t + 1, (t + 1) % 2)
            c = wid + t * _NW

            @pl.when(c < n_chunks)
            def _():
                pltpu.make_async_copy(
                    a_hbm.at[idx_b[par]], rows_b[par], sem_b[par]).wait()
                rows_v = rows_b[par]

                @plsc.parallel_loop(0, _GROUPS, unroll=2)
                def grp_body(g):
                    rows16 = g * 16 + lanes
                    r = g // 8
                    o = (g % 8) * 16
                    for f in range(13):
                        v = plsc.load_gather(
                            rows_v, [rows16, jnp.full((16,), f, jnp.int32)])
                        rowsT_v[f, r, pl.ds(o, 16)] = v

                pltpu.sync_copy(
                    rowsT_v,
                    out_hbm.at[:, c // _CPM,
                               pl.ds((c % _CPM) * _CH_ROWS, _CH_ROWS), :])

  return _gather_body


@functools.cache
def _sc_gather(n_pairs):
    return pl.kernel(
        _make_gather_body(n_pairs),
        out_type=[
            jax.ShapeDtypeStruct((13, n_pairs // 6400, 50, 128), jnp.float32),
            jax.ShapeDtypeStruct((13, n_pairs // 6400, 50, 128), jnp.float32),
        ],
        mesh=plsc.VectorSubcoreMesh(core_axis_name="c", subcore_axis_name="s"),
        scratch_types=[
            pltpu.VMEM((_CH,), jnp.int32),
            pltpu.VMEM((_CH,), jnp.int32),
            pltpu.VMEM((_CH, 16), jnp.float32),
            pltpu.VMEM((_CH, 16), jnp.float32),
            pltpu.VMEM((13, _CH_ROWS, 128), jnp.float32),
            pltpu.SemaphoreType.DMA,
            pltpu.SemaphoreType.DMA,
        ],
        compiler_params=pltpu.CompilerParams(
            use_tc_tiling_on_sc=False, needs_layout_passes=False),
    )


def _energy_body(box_ref, binv_ref, gi_ref, gj_ref, out_ref):
    giT = gi_ref
    gjT = gj_ref

    def row(arr, r):
        return arr[r, 0, :, :]

    cix, ciy, ciz = row(giT, 0), row(giT, 1), row(giT, 2)
    cjx, cjy, cjz = row(gjT, 0), row(gjT, 1), row(gjT, 2)

    def bf(v):
        return v.astype(jnp.bfloat16).astype(jnp.float32)

    d0, d1, d2 = bf(cjx - cix), bf(cjy - ciy), bf(cjz - ciz)

    def mat(ref, m, k):
        return bf(ref[m, k])

    s0 = d0 * mat(binv_ref, 0, 0) + d1 * mat(binv_ref, 1, 0) + d2 * mat(binv_ref, 2, 0)
    s1 = d0 * mat(binv_ref, 0, 1) + d1 * mat(binv_ref, 1, 1) + d2 * mat(binv_ref, 2, 1)
    s2 = d0 * mat(binv_ref, 0, 2) + d1 * mat(binv_ref, 1, 2) + d2 * mat(binv_ref, 2, 2)
    s0 = bf(s0 - jnp.round(s0))
    s1 = bf(s1 - jnp.round(s1))
    s2 = bf(s2 - jnp.round(s2))
    x = s0 * mat(box_ref, 0, 0) + s1 * mat(box_ref, 1, 0) + s2 * mat(box_ref, 2, 0)
    y = s0 * mat(box_ref, 0, 1) + s1 * mat(box_ref, 1, 1) + s2 * mat(box_ref, 2, 1)
    z = s0 * mat(box_ref, 0, 2) + s1 * mat(box_ref, 1, 2) + s2 * mat(box_ref, 2, 2)

    dr2 = x * x + y * y + z * z
    dr = jnp.sqrt(dr2)
    drInv = 1.0 / dr
    mask = dr <= RANK2_CUTOFF

    u = ALPHA * dr
    t = 1.0 / (1.0 + 0.3275911 * u)
    exp2u = jnp.exp(-u * u)
    erfc_u = (
        t
        * (0.254829592
           + t * (-0.284496736
                  + t * (1.421413741
                         + t * (-1.453152027 + t * 1.061405429))))
        * exp2u
    )
    u2 = u * u
    u3 = u2 * u
    u5 = u3 * u2
    u7 = u5 * u2
    prefpi = 2.0 / math.sqrt(math.pi)
    g = prefpi * exp2u
    f1 = erfc_u
    f3 = erfc_u + g * u
    f5 = erfc_u + g * ((3.0 * u + 2.0 * u3) / 3.0)
    f7 = erfc_u + g * ((15.0 * u + 10.0 * u3 + 4.0 * u5) / 15.0)
    f9 = erfc_u + g * ((8.0 * u7 + 28.0 * u5 + 70.0 * u3 + 105.0 * u) / 105.0)

    drInv2 = drInv * drInv
    D1 = drInv * f1
    drInv3 = drInv2 * drInv
    drInv5 = drInv3 * drInv2
    drInv7 = drInv5 * drInv2
    drInv9 = drInv7 * drInv2
    D3 = drInv3 * f3
    D5 = drInv5 * f5
    D7 = drInv7 * f7
    D9 = drInv9 * f9

    x2, y2, z2 = x * x, y * y, z * z
    xy, xz, yz = x * y, x * z, y * z

    tx, ty, tz = -x * D3, -y * D3, -z * D3
    txx = 3.0 * x2 * D5 - D3
    txy = 3.0 * xy * D5
    txz = 3.0 * xz * D5
    tyy = 3.0 * y2 * D5 - D3
    tyz = 3.0 * yz * D5
    tzz = 3.0 * z2 * D5 - D3
    txxx = -15.0 * x2 * x * D7 + 9.0 * x * D5
    txxy = -15.0 * x2 * y * D7 + 3.0 * y * D5
    txxz = -15.0 * x2 * z * D7 + 3.0 * z * D5
    tyyy = -15.0 * y2 * y * D7 + 9.0 * y * D5
    tyyx = -15.0 * y2 * x * D7 + 3.0 * x * D5
    tyyz = -15.0 * y2 * z * D7 + 3.0 * z * D5
    tzzz = -15.0 * z2 * z * D7 + 9.0 * z * D5
    tzzx = -15.0 * z2 * x * D7 + 3.0 * x * D5
    tzzy = -15.0 * z2 * y * D7 + 3.0 * y * D5
    txyz = -15.0 * x * y * z * D7
    txxxx = 105.0 * x2 * x2 * D9 - 90.0 * x2 * D7 + 9.0 * D5
    txxxy = 105.0 * x2 * xy * D9 - 45.0 * xy * D7
    txxxz = 105.0 * x2 * xz * D9 - 45.0 * xz * D7
    txxyy = 105.0 * x2 * y2 * D9 - 15.0 * (x2 + y2) * D7 + 3.0 * D5
    txxzz = 105.0 * x2 * z2 * D9 - 15.0 * (x2 + z2) * D7 + 3.0 * D5
    txxyz = 105.0 * x2 * yz * D9 - 15.0 * yz * D7
    tyyyy = 105.0 * y2 * y2 * D9 - 90.0 * y2 * D7 + 9.0 * D5
    tyyyx = 105.0 * y2 * xy * D9 - 45.0 * xy * D7
    tyyyz = 105.0 * y2 * yz * D9 - 45.0 * yz * D7
    tyyzz = 105.0 * y2 * z2 * D9 - 15.0 * (y2 + z2) * D7 + 3.0 * D5
    tyyxz = 105.0 * y2 * xz * D9 - 15.0 * xz * D7
    tzzzz = 105.0 * z2 * z2 * D9 - 90.0 * z2 * D7 + 9.0 * D5
    tzzzx = 105.0 * z2 * xz * D9 - 45.0 * xz * D7
    tzzzy = 105.0 * z2 * yz * D9 - 45.0 * yz * D7
    tzzxy = 105.0 * z2 * xy * D9 - 15.0 * xy * D7

    qi = row(giT, 3)
    pix, piy, piz = row(giT, 4), row(giT, 5), row(giT, 6)
    Qi0, Qi1, Qi2 = row(giT, 7), row(giT, 8), row(giT, 9)
    Qi3, Qi4, Qi5 = row(giT, 10), row(giT, 11), row(giT, 12)
    qj = row(gjT, 3)
    pjx, pjy, pjz = row(gjT, 4), row(gjT, 5), row(gjT, 6)
    Qj0, Qj1, Qj2 = row(gjT, 7), row(gjT, 8), row(gjT, 9)
    Qj3, Qj4, Qj5 = row(gjT, 10), row(gjT, 11), row(gjT, 12)

    r0 = (D1 * qj - tx * pjx - ty * pjy - tz * pjz
          + txx * Qj0 + txy * Qj1 + txz * Qj2 + tyy * Qj3 + tyz * Qj4 + tzz * Qj5)
    r1 = (tx * qj - txx * pjx - txy * pjy - txz * pjz
          + txxx * Qj0 + txxy * Qj1 + txxz * Qj2 + tyyx * Qj3 + txyz * Qj4 + tzzx * Qj5)
    r2 = (ty * qj - txy * pjx - tyy * pjy - tyz * pjz
          + txxy * Qj0 + tyyx * Qj1 + txyz * Qj2 + tyyy * Qj3 + tyyz * Qj4 + tzzy * Qj5)
    r3 = (tz * qj - txz * pjx - tyz * pjy - tzz * pjz
          + txxz * Qj0 + txyz * Qj1 + tzzx * Qj2 + tyyz * Qj3 + tzzy * Qj4 + tzzz * Qj5)
    r4 = (txx * qj - txxx * pjx - txxy * pjy - txxz * pjz
          + txxxx * Qj0 + txxxy * Qj1 + txxxz * Qj2 + txxyy * Qj3 + txxyz * Qj4 + txxzz * Qj5)
    r5 = (txy * qj - txxy * pjx - tyyx * pjy - txyz * pjz
          + txxxy * Qj0 + txxyy * Qj1 + txxyz * Qj2 + tyyyx * Qj3 + tyyxz * Qj4 + tzzxy * Qj5)
    r6 = (txz * qj - txxz * pjx - txyz * pjy - tzzx * pjz
          + txxxz * Qj0 + txxyz * Qj1 + txxzz * Qj2 + tyyxz * Qj3 + tzzxy * Qj4 + tzzzx * Qj5)
    r7 = (tyy * qj - tyyx * pjx - tyyy * pjy - tyyz * pjz
          + txxyy * Qj0 + tyyyx * Qj1 + tyyxz * Qj2 + tyyyy * Qj3 + tyyyz * Qj4 + tyyzz * Qj5)
    r8 = (tyz * qj - txyz * pjx - tyyz * pjy - tzzy * pjz
          + txxyz * Qj0 + tyyxz * Qj1 + tzzxy * Qj2 + tyyyz * Qj3 + tyyzz * Qj4 + tzzzy * Qj5)
    r9 = (tzz * qj - tzzx * pjx - tzzy * pjy - tzzz * pjz
          + txxzz * Qj0 + tzzxy * Qj1 + tzzzx * Qj2 + tyyzz * Qj3 + tzzzy * Qj4 + tzzzz * Qj5)

    ene = (qi * r0 + pix * r1 + piy * r2 + piz * r3
           + Qi0 * r4 + Qi1 * r5 + Qi2 * r6
           + Qi3 * r7 + Qi4 * r8 + Qi5 * r9)
    ene = jnp.where(mask, ene, 0.0)
    part = jnp.sum(ene)

    @pl.when(pl.program_id(0) == 0)
    def _():
        out_ref[0, 0] = 0.0

    out_ref[0, 0] += part


_TC_BLK = 6400


def _tc_energy(box, box_inv, gi, gj):
    return pl.pallas_call(
        _energy_body,
        grid=(gi.shape[1],),
        in_specs=[
            pl.BlockSpec(memory_space=pltpu.SMEM),
            pl.BlockSpec(memory_space=pltpu.SMEM),
            pl.BlockSpec((13, 1, 50, 128), lambda i: (0, i, 0, 0)),
            pl.BlockSpec((13, 1, 50, 128), lambda i: (0, i, 0, 0)),
        ],
        out_specs=pl.BlockSpec(memory_space=pltpu.SMEM),
        out_shape=jax.ShapeDtypeStruct((1, 1), jnp.float32),
    )(box, box_inv, gi, gj)


def kernel(coords, box, pairs, q, p, t):
    box_inv = jnp.linalg.inv(box)
    a_tab = jnp.concatenate(
        [
            coords,
            q[:, None],
            p,
            t[:, 0, 0][:, None] / 3,
            (t[:, 0, 1] + t[:, 1, 0])[:, None] / 3,
            (t[:, 0, 2] + t[:, 2, 0])[:, None] / 3,
            t[:, 1, 1][:, None] / 3,
            (t[:, 1, 2] + t[:, 2, 1])[:, None] / 3,
            t[:, 2, 2][:, None] / 3,
            jnp.zeros((coords.shape[0], 3), jnp.float32),
        ],
        axis=1,
    )
    src = pairs[:, 0]
    dst = pairs[:, 1]
    h = 396800
    gi1, gj1 = _sc_gather(h)(a_tab, src[:h], dst[:h])
    gi2, gj2 = _sc_gather(N_PAIRS_P - h)(a_tab, src[h:], dst[h:])
    out1 = _tc_energy(box, box_inv, gi1, gj1)
    out2 = _tc_energy(box, box_inv, gi2, gj2)
    return PREF * (out1[0, 0] + out2[0, 0])
